# Initial kernel scaffold; baseline (speedup 1.0000x reference)
import functools

import jax
import jax.numpy as jnp
from jax import lax
from jax.experimental import pallas as pl
from jax.experimental.pallas import tpu as pltpu
from jax.experimental.pallas import tpu_sc as plsc

N = 10000
E = 320000
D = 128
NC = 2
NS = 16
NW = NC * NS
LANES = 16

EDGES_PER_W = E // NW
TILE_E = 80
ROWS_PER_S = N // NS
ZCHUNK = 125


def _sc_aggregate(x, src, dst, vals):
    mesh = plsc.VectorSubcoreMesh(core_axis_name="c", subcore_axis_name="s")

    @functools.partial(
        pl.kernel,
        mesh=mesh,
        out_type=jax.ShapeDtypeStruct((NC, N, D), jnp.float32),
        scratch_types=[
            pltpu.VMEM((ZCHUNK, D), jnp.float32),
            pltpu.VMEM((TILE_E,), jnp.int32),
            pltpu.VMEM((TILE_E,), jnp.int32),
            pltpu.VMEM((TILE_E,), jnp.float32),
            pltpu.VMEM((TILE_E, D), jnp.float32),
            pltpu.VMEM_SHARED((N, D), jnp.float32),
            pltpu.SemaphoreType.DMA,
        ],
    )
    def agg_kernel(x_hbm, src_hbm, dst_hbm, vals_hbm, out_hbm,
                   zbuf, src_v, dst_v, vals_v, rows_v, acc, sem):
        c = lax.axis_index("c")
        s = lax.axis_index("s")
        wid = c * NS + s

        @pl.loop(0, ZCHUNK)
        def _(i):
            @pl.loop(0, D, step=LANES)
            def _(j):
                zbuf[i, pl.ds(j, LANES)] = jnp.zeros((LANES,), jnp.float32)

        @pl.loop(0, ROWS_PER_S, step=ZCHUNK)
        def _(k):
            pltpu.sync_copy(zbuf, acc.at[pl.ds(s * ROWS_PER_S + k, ZCHUNK)])

        plsc.subcore_barrier()

        base_w = wid * EDGES_PER_W

        @pl.loop(0, EDGES_PER_W, step=TILE_E)
        def _(t):
            base = base_w + t
            pltpu.sync_copy(src_hbm.at[pl.ds(base, TILE_E)], src_v)
            pltpu.sync_copy(dst_hbm.at[pl.ds(base, TILE_E)], dst_v)
            pltpu.sync_copy(vals_hbm.at[pl.ds(base, TILE_E)], vals_v)
            pltpu.async_copy(x_hbm.at[src_v], rows_v, sem).wait()

            @pl.loop(0, TILE_E // LANES)
            def _(g):
                vv = vals_v[pl.ds(g * LANES, LANES)]
                for r in range(LANES):
                    row = g * LANES + r
                    sp = jnp.broadcast_to(vv[r], (LANES,))
                    for j in range(0, D, LANES):
                        rows_v[row, pl.ds(j, LANES)] = (
                            rows_v[row, pl.ds(j, LANES)] * sp)

            pltpu.sync_copy(rows_v, acc.at[dst_v], add=True)

        plsc.subcore_barrier()

        pltpu.sync_copy(
            acc.at[pl.ds(s * ROWS_PER_S, ROWS_PER_S)],
            out_hbm.at[c, pl.ds(s * ROWS_PER_S, ROWS_PER_S)])

    return agg_kernel(x, src, dst, vals)


_MM_BLK = 2000


def _mm_body(p_ref, w_ref, b_ref, o_ref):
    summed = p_ref[0] + p_ref[1]
    o_ref[...] = lax.dot_general(
        summed, w_ref[...], (((1,), (0,)), ((), ())),
        preferred_element_type=jnp.float32,
        precision=lax.Precision.HIGHEST,
    ) + b_ref[...]


def _tc_matmul(parts, W, b2d):
    return pl.pallas_call(
        _mm_body,
        grid=(N // _MM_BLK,),
        in_specs=[
            pl.BlockSpec((NC, _MM_BLK, D), lambda i: (0, i, 0)),
            pl.BlockSpec((D, D), lambda i: (0, 0)),
            pl.BlockSpec((1, D), lambda i: (0, 0)),
        ],
        out_specs=pl.BlockSpec((_MM_BLK, D), lambda i: (i, 0)),
        out_shape=jax.ShapeDtypeStruct((N, D), jnp.float32),
    )(parts, W, b2d)


def kernel(x, edge_index, edge_vals, W, b):
    src = edge_index[0]
    dst = edge_index[1]
    parts = _sc_aggregate(x, src, dst, edge_vals)
    return _tc_matmul(parts, W, b.reshape(1, D))


# speedup vs baseline: 1.9552x; 1.9552x over previous
import functools

import jax
import jax.numpy as jnp
from jax import lax
from jax.experimental import pallas as pl
from jax.experimental.pallas import tpu as pltpu
from jax.experimental.pallas import tpu_sc as plsc

N = 10000
E = 320000
D = 128
DH = D // 2
NC = 2
NS = 16
LANES = 16

EDGES_PER_S = E // NS
TILE_E = 80
ROWS_PER_S = 632
NP = ROWS_PER_S * NS


def _sc_aggregate(x2, src, dst, vals):
    mesh = plsc.VectorSubcoreMesh(core_axis_name="c", subcore_axis_name="s")

    @functools.partial(
        pl.kernel,
        mesh=mesh,
        compiler_params=pltpu.CompilerParams(use_tc_tiling_on_sc=False),
        out_type=jax.ShapeDtypeStruct((NC, NP, DH), jnp.float32),
        scratch_types=[
            pltpu.VMEM((ROWS_PER_S, DH), jnp.float32),
            pltpu.VMEM((TILE_E,), jnp.int32),
            pltpu.VMEM((TILE_E,), jnp.int32),
            pltpu.VMEM((TILE_E,), jnp.int32),
            pltpu.VMEM((TILE_E,), jnp.float32),
            pltpu.VMEM((TILE_E, DH), jnp.float32),
            pltpu.VMEM_SHARED((NP, DH), jnp.float32),
            pltpu.SemaphoreType.DMA,
        ],
    )
    def agg_kernel(x_hbm, src_hbm, dst_hbm, vals_hbm, out_hbm,
                   zbuf, src_v, gidx_v, dst_v, vals_v, rows_v, acc, sem):
        c = lax.axis_index("c")
        s = lax.axis_index("s")

        @pl.loop(0, ROWS_PER_S)
        def _(i):
            @pl.loop(0, DH, step=LANES)
            def _(j):
                zbuf[i, pl.ds(j, LANES)] = jnp.zeros((LANES,), jnp.float32)

        pltpu.sync_copy(zbuf, acc.at[pl.ds(s * ROWS_PER_S, ROWS_PER_S)])

        plsc.subcore_barrier()

        base_s = s * EDGES_PER_S

        @pl.loop(0, EDGES_PER_S, step=TILE_E)
        def _(t):
            base = base_s + t
            pltpu.sync_copy(src_hbm.at[pl.ds(base, TILE_E)], src_v)
            pltpu.sync_copy(dst_hbm.at[pl.ds(base, TILE_E)], dst_v)
            pltpu.sync_copy(vals_hbm.at[pl.ds(base, TILE_E)], vals_v)

            @pl.loop(0, TILE_E, step=LANES)
            def _(i):
                sv = src_v[pl.ds(i, LANES)]
                gidx_v[pl.ds(i, LANES)] = sv * 2 + c

            pltpu.async_copy(x_hbm.at[gidx_v], rows_v, sem).wait()

            @pl.loop(0, TILE_E // LANES)
            def _(g):
                vv = vals_v[pl.ds(g * LANES, LANES)]
                for r in range(LANES):
                    row = g * LANES + r
                    sp = jnp.broadcast_to(vv[r], (LANES,))
                    for j in range(0, DH, LANES):
                        rows_v[row, pl.ds(j, LANES)] = (
                            rows_v[row, pl.ds(j, LANES)] * sp)

            pltpu.sync_copy(rows_v, acc.at[dst_v], add=True)

        plsc.subcore_barrier()

        pltpu.sync_copy(
            acc.at[pl.ds(s * ROWS_PER_S, ROWS_PER_S)],
            out_hbm.at[c, pl.ds(s * ROWS_PER_S, ROWS_PER_S)])

    return agg_kernel(x2, src, dst, vals)


_MM_BLK = 2000


def _mm_body(p_ref, w_ref, b_ref, o_ref):
    o_ref[...] = (
        lax.dot_general(
            p_ref[0], w_ref[0:DH, :], (((1,), (0,)), ((), ())),
            preferred_element_type=jnp.float32,
            precision=lax.Precision.HIGHEST)
        + lax.dot_general(
            p_ref[1], w_ref[DH:D, :], (((1,), (0,)), ((), ())),
            preferred_element_type=jnp.float32,
            precision=lax.Precision.HIGHEST)
        + b_ref[...])


def _tc_matmul(parts, W, b2d):
    return pl.pallas_call(
        _mm_body,
        grid=(N // _MM_BLK,),
        in_specs=[
            pl.BlockSpec((NC, _MM_BLK, DH), lambda i: (0, i, 0)),
            pl.BlockSpec((D, D), lambda i: (0, 0)),
            pl.BlockSpec((1, D), lambda i: (0, 0)),
        ],
        out_specs=pl.BlockSpec((_MM_BLK, D), lambda i: (i, 0)),
        out_shape=jax.ShapeDtypeStruct((N, D), jnp.float32),
    )(parts, W, b2d)


def kernel(x, edge_index, edge_vals, W, b):
    src = edge_index[0]
    dst = edge_index[1]
    x2 = x.reshape(2 * N, DH)
    parts = _sc_aggregate(x2, src, dst, edge_vals)
    return _tc_matmul(parts, W, b.reshape(1, D))
